# baseline (speedup 1.0000x reference)
import functools

import jax
import jax.numpy as jnp
from jax import lax
from jax.experimental import pallas as pl
from jax.experimental.pallas import tpu as pltpu
from jax.experimental.pallas import tpu_sc as plsc

BATCH = 16384
FACTOR = 32
NC = 2
NS = 16
NW = NC * NS
B_PER_W = BATCH // NW
CHUNK = 128
NCHUNK = B_PER_W // CHUNK


def _gmf_body(uidx_hbm, iidx_hbm, utab_hbm, itab_hbm, out_hbm,
              uidx_v, iidx_v, urows_v, irows_v, sem):
    wid = lax.axis_index("s") * NC + lax.axis_index("c")
    blk = wid * NCHUNK

    pltpu.sync_copy(uidx_hbm.at[pl.ds(blk, NCHUNK), :], uidx_v)
    pltpu.sync_copy(iidx_hbm.at[pl.ds(blk, NCHUNK), :], iidx_v)

    copies = []
    for j in range(NCHUNK):
        copies.append(pltpu.async_copy(
            utab_hbm.at[uidx_v.at[j]],
            urows_v.at[pl.ds(j * CHUNK, CHUNK), :], sem))
        copies.append(pltpu.async_copy(
            itab_hbm.at[iidx_v.at[j]],
            irows_v.at[pl.ds(j * CHUNK, CHUNK), :], sem))
    for c in copies:
        c.wait()

    def body(r, _):
        for c in range(0, FACTOR, 16):
            urows_v[r, pl.ds(c, 16)] = (
                urows_v[r, pl.ds(c, 16)] * irows_v[r, pl.ds(c, 16)])
        return ()

    lax.fori_loop(0, B_PER_W, body, (), unroll=4)

    pltpu.sync_copy(urows_v, out_hbm.at[pl.ds(wid * B_PER_W, B_PER_W), :])


@jax.jit
def _gmf(uidx2, iidx2, user_table, item_table):
    mesh = plsc.VectorSubcoreMesh(core_axis_name="c", subcore_axis_name="s")
    kfn = functools.partial(
        pl.kernel,
        mesh=mesh,
        compiler_params=pltpu.CompilerParams(use_tc_tiling_on_sc=False),
        out_type=jax.ShapeDtypeStruct((BATCH, FACTOR), jnp.float32),
        scratch_types=[
            pltpu.VMEM((NCHUNK, CHUNK), jnp.int32),
            pltpu.VMEM((NCHUNK, CHUNK), jnp.int32),
            pltpu.VMEM((B_PER_W, FACTOR), jnp.float32),
            pltpu.VMEM((B_PER_W, FACTOR), jnp.float32),
            pltpu.SemaphoreType.DMA,
        ],
    )(_gmf_body)
    return kfn(uidx2, iidx2, user_table, item_table)


def kernel(user_indices, item_indices, user_table, item_table):
    uidx2 = user_indices.astype(jnp.int32).reshape(BATCH // CHUNK, CHUNK)
    iidx2 = item_indices.astype(jnp.int32).reshape(BATCH // CHUNK, CHUNK)
    return _gmf(uidx2, iidx2, user_table, item_table)
